# baseline (speedup 1.0000x reference)
import functools

import jax
import jax.numpy as jnp
from jax import lax
from jax.experimental import pallas as pl
from jax.experimental.pallas import tpu as pltpu
from jax.experimental.pallas import tpu_sc as plsc

B = 16384
K = 64
ROW = 128
NUM_WORKERS = 32
BPW = B // NUM_WORKERS
CHUNK = 256
LANES = 16

def _hi_to_f32(h):
    return plsc.bitcast(h, jnp.float32)


def _sc_kernel_body(f0_hbm, f1_hbm, f2_hbm, i0_hbm, i1_hbm, i2_hbm, out_hbm,
                    i0_v, i1_v, i2_v, r0_v, r1_v, r2_v, out_v, sem):
    wid = lax.axis_index("s") * 2 + lax.axis_index("c")
    base = wid * BPW

    pltpu.sync_copy(i0_hbm.at[pl.ds(base, BPW)], i0_v)
    pltpu.sync_copy(i1_hbm.at[pl.ds(base, BPW)], i1_v)
    pltpu.sync_copy(i2_hbm.at[pl.ds(base, BPW)], i2_v)

    lane_iota = lax.iota(jnp.int32, LANES)
    hi_cols = [jnp.int32(2) * (jnp.int32(16 * j) + lane_iota) + jnp.int32(1)
               for j in range(K // LANES)]

    for c in range(BPW // CHUNK):
        sl_c = pl.ds(c * CHUNK, CHUNK)
        c0 = pltpu.async_copy(f0_hbm.at[i0_v.at[sl_c]], r0_v, sem)
        c1 = pltpu.async_copy(f1_hbm.at[i1_v.at[sl_c]], r1_v, sem)
        c2 = pltpu.async_copy(f2_hbm.at[i2_v.at[sl_c]], r2_v, sem)
        c0.wait()
        c1.wait()
        c2.wait()

        def body(g, carry):
            vec = jnp.zeros((LANES,), jnp.float32)
            gbase = g * jnp.int32(LANES)
            for l in range(LANES):
                b = jnp.broadcast_to(gbase + jnp.int32(l), (LANES,))
                acc = None
                for j in range(K // LANES):
                    v0 = _hi_to_f32(plsc.load_gather(r0_v, [b, hi_cols[j]]))
                    v1 = _hi_to_f32(plsc.load_gather(r1_v, [b, hi_cols[j]]))
                    v2 = _hi_to_f32(plsc.load_gather(r2_v, [b, hi_cols[j]]))
                    p = v0 * v1 * v2
                    acc = p if acc is None else acc + p
                vec = jnp.where(lane_iota == jnp.int32(l), jnp.sum(acc), vec)
            out_v[pl.ds(jnp.int32(c * CHUNK) + gbase, LANES)] = vec
            return carry

        lax.fori_loop(jnp.int32(0), jnp.int32(CHUNK // LANES), body,
                      jnp.int32(0))

    pltpu.sync_copy(out_v, out_hbm.at[pl.ds(base, BPW)])


@jax.jit
def _run(f0, f1, f2, i0, i1, i2):
    mesh = plsc.VectorSubcoreMesh(core_axis_name="c", subcore_axis_name="s")
    kern = functools.partial(
        pl.kernel,
        out_type=jax.ShapeDtypeStruct((B,), jnp.float32),
        mesh=mesh,
        scratch_types=[
            pltpu.VMEM((BPW,), jnp.int32),
            pltpu.VMEM((BPW,), jnp.int32),
            pltpu.VMEM((BPW,), jnp.int32),
            pltpu.VMEM((CHUNK, ROW), jnp.int32),
            pltpu.VMEM((CHUNK, ROW), jnp.int32),
            pltpu.VMEM((CHUNK, ROW), jnp.int32),
            pltpu.VMEM((BPW,), jnp.float32),
            pltpu.SemaphoreType.DMA,
        ],
        compiler_params=pltpu.CompilerParams(needs_layout_passes=False),
    )(_sc_kernel_body)
    return kern(f0, f1, f2, i0, i1, i2)


def kernel(indices, F0, F1, F2):
    idx = indices.astype(jnp.int32)
    f0 = lax.bitcast_convert_type(F0, jnp.int32).reshape(F0.shape[0], ROW)
    f1 = lax.bitcast_convert_type(F1, jnp.int32).reshape(F1.shape[0], ROW)
    f2 = lax.bitcast_convert_type(F2, jnp.int32).reshape(F2.shape[0], ROW)
    out = _run(f0, f1, f2, idx[:, 0], idx[:, 1], idx[:, 2])
    return out.astype(jnp.float64)


# speedup vs baseline: 1.2997x; 1.2997x over previous
import functools

import jax
import jax.numpy as jnp
from jax import lax
from jax.experimental import pallas as pl
from jax.experimental.pallas import tpu as pltpu
from jax.experimental.pallas import tpu_sc as plsc

B = 16384
K = 64
NUM_WORKERS = 32
BPW = B // NUM_WORKERS
LANES = 16


def _sc_kernel_body(f0_hbm, f1_hbm, f2_hbm, i0_hbm, i1_hbm, i2_hbm, out_hbm,
                    i0_v, i1_v, i2_v, r0_v, r1_v, r2_v, out_v, sem):
    wid = lax.axis_index("s") * 2 + lax.axis_index("c")
    base = wid * BPW

    pltpu.sync_copy(i0_hbm.at[pl.ds(base, BPW)], i0_v)
    pltpu.sync_copy(i1_hbm.at[pl.ds(base, BPW)], i1_v)
    pltpu.sync_copy(i2_hbm.at[pl.ds(base, BPW)], i2_v)

    c0 = pltpu.async_copy(f0_hbm.at[i0_v], r0_v, sem)
    c1 = pltpu.async_copy(f1_hbm.at[i1_v], r1_v, sem)
    c2 = pltpu.async_copy(f2_hbm.at[i2_v], r2_v, sem)
    c0.wait()
    c1.wait()
    c2.wait()

    lane_iota = lax.iota(jnp.int32, LANES)

    def body(g, carry):
        vec = jnp.zeros((LANES,), jnp.float32)
        gbase = g * jnp.int32(LANES)
        for l in range(LANES):
            b = gbase + jnp.int32(l)
            acc = None
            for j in range(K // LANES):
                sl = pl.ds(j * LANES, LANES)
                p = r0_v[b, sl] * r1_v[b, sl] * r2_v[b, sl]
                acc = p if acc is None else acc + p
            vec = jnp.where(lane_iota == jnp.int32(l), jnp.sum(acc), vec)
        out_v[pl.ds(gbase, LANES)] = vec
        return carry

    lax.fori_loop(jnp.int32(0), jnp.int32(BPW // LANES), body, jnp.int32(0))

    pltpu.sync_copy(out_v, out_hbm.at[pl.ds(base, BPW)])


@jax.jit
def _run(f0, f1, f2, i0, i1, i2):
    mesh = plsc.VectorSubcoreMesh(core_axis_name="c", subcore_axis_name="s")
    kern = functools.partial(
        pl.kernel,
        out_type=jax.ShapeDtypeStruct((B,), jnp.float32),
        mesh=mesh,
        scratch_types=[
            pltpu.VMEM((BPW,), jnp.int32),
            pltpu.VMEM((BPW,), jnp.int32),
            pltpu.VMEM((BPW,), jnp.int32),
            pltpu.VMEM((BPW, K), jnp.float32),
            pltpu.VMEM((BPW, K), jnp.float32),
            pltpu.VMEM((BPW, K), jnp.float32),
            pltpu.VMEM((BPW,), jnp.float32),
            pltpu.SemaphoreType.DMA,
        ],
        compiler_params=pltpu.CompilerParams(
            needs_layout_passes=False, use_tc_tiling_on_sc=False),
    )(_sc_kernel_body)
    return kern(f0, f1, f2, i0, i1, i2)


def _hi_plane(f):
    pair = lax.bitcast_convert_type(f, jnp.float32)
    return pair[:, :, 1]


def kernel(indices, F0, F1, F2):
    idx = indices.astype(jnp.int32)
    out = _run(_hi_plane(F0), _hi_plane(F1), _hi_plane(F2),
               idx[:, 0], idx[:, 1], idx[:, 2])
    return out.astype(jnp.float64)


# speedup vs baseline: 1.4184x; 1.0914x over previous
import functools

import jax
import jax.numpy as jnp
from jax import lax
from jax.experimental import pallas as pl
from jax.experimental.pallas import tpu as pltpu
from jax.experimental.pallas import tpu_sc as plsc

B = 16384
K = 64
NUM_WORKERS = 32
BPW = B // NUM_WORKERS
LANES = 16


def _sc_kernel_body(f0_hbm, f1_hbm, f2_hbm, i0_hbm, i1_hbm, i2_hbm, out_hbm,
                    i0_v, i1_v, i2_v, r0_v, r1_v, r2_v, out_v, sem):
    wid = lax.axis_index("s") * 2 + lax.axis_index("c")
    base = wid * BPW

    pltpu.sync_copy(i0_hbm.at[pl.ds(base, BPW)], i0_v)
    pltpu.sync_copy(i1_hbm.at[pl.ds(base, BPW)], i1_v)
    pltpu.sync_copy(i2_hbm.at[pl.ds(base, BPW)], i2_v)

    c0 = pltpu.async_copy(f0_hbm.at[i0_v], r0_v, sem)
    c1 = pltpu.async_copy(f1_hbm.at[i1_v], r1_v, sem)
    c2 = pltpu.async_copy(f2_hbm.at[i2_v], r2_v, sem)
    c0.wait()
    c1.wait()
    c2.wait()

    lane_iota = lax.iota(jnp.int32, LANES)

    def body(g, carry):
        vec = jnp.zeros((LANES,), jnp.float32)
        gbase = g * jnp.int32(LANES)
        for l in range(LANES):
            b = gbase + jnp.int32(l)
            acc = None
            for j in range(K // LANES):
                sl = pl.ds(j * LANES, LANES)
                p = (plsc.bitcast(r0_v[b, sl], jnp.float32)
                     * plsc.bitcast(r1_v[b, sl], jnp.float32)
                     * plsc.bitcast(r2_v[b, sl], jnp.float32))
                acc = p if acc is None else acc + p
            vec = jnp.where(lane_iota == jnp.int32(l), jnp.sum(acc), vec)
        out_v[pl.ds(gbase, LANES)] = vec
        return carry

    lax.fori_loop(jnp.int32(0), jnp.int32(BPW // LANES), body, jnp.int32(0))

    pltpu.sync_copy(out_v, out_hbm.at[pl.ds(base, BPW)])


@jax.jit
def _run(f0, f1, f2, i0, i1, i2):
    mesh = plsc.VectorSubcoreMesh(core_axis_name="c", subcore_axis_name="s")
    kern = functools.partial(
        pl.kernel,
        out_type=jax.ShapeDtypeStruct((B,), jnp.float32),
        mesh=mesh,
        scratch_types=[
            pltpu.VMEM((BPW,), jnp.int32),
            pltpu.VMEM((BPW,), jnp.int32),
            pltpu.VMEM((BPW,), jnp.int32),
            pltpu.VMEM((BPW, K), jnp.int32),
            pltpu.VMEM((BPW, K), jnp.int32),
            pltpu.VMEM((BPW, K), jnp.int32),
            pltpu.VMEM((BPW,), jnp.float32),
            pltpu.SemaphoreType.DMA,
        ],
        compiler_params=pltpu.CompilerParams(
            needs_layout_passes=False, use_tc_tiling_on_sc=False),
    )(_sc_kernel_body)
    return kern(f0, f1, f2, i0, i1, i2)


def _hi_plane(f):
    return lax.bitcast_convert_type(f.astype(jnp.float32), jnp.int32)


def kernel(indices, F0, F1, F2):
    idx = indices.astype(jnp.int32)
    out = _run(_hi_plane(F0), _hi_plane(F1), _hi_plane(F2),
               idx[:, 0], idx[:, 1], idx[:, 2])
    return out.astype(jnp.float64)


# speedup vs baseline: 2.8180x; 1.9867x over previous
import functools

import jax
import jax.numpy as jnp
from jax import lax
from jax.experimental import pallas as pl
from jax.experimental.pallas import tpu as pltpu
from jax.experimental.pallas import tpu_sc as plsc

B = 16384
K = 64
NUM_WORKERS = 32
BPW = B // NUM_WORKERS
LANES = 16


def _sc_kernel_body(f0_hbm, f1_hbm, f2_hbm, i0_hbm, i1_hbm, i2_hbm, out_hbm,
                    i0_v, i1_v, i2_v, r0_v, r1_v, r2_v, out_v, sem):
    wid = lax.axis_index("s") * 2 + lax.axis_index("c")
    base = wid * BPW

    pltpu.sync_copy(i0_hbm.at[pl.ds(base, BPW)], i0_v)
    pltpu.sync_copy(i1_hbm.at[pl.ds(base, BPW)], i1_v)
    pltpu.sync_copy(i2_hbm.at[pl.ds(base, BPW)], i2_v)

    c0 = pltpu.async_copy(f0_hbm.at[i0_v], r0_v, sem)
    c1 = pltpu.async_copy(f1_hbm.at[i1_v], r1_v, sem)
    c2 = pltpu.async_copy(f2_hbm.at[i2_v], r2_v, sem)
    c0.wait()
    c1.wait()
    c2.wait()

    lane_iota = lax.iota(jnp.int32, LANES)

    def body(g, carry):
        vec = jnp.zeros((LANES,), jnp.float32)
        gbase = g * jnp.int32(LANES)
        for l in range(LANES):
            b = gbase + jnp.int32(l)
            acc = None
            for j in range(K // LANES):
                sl = pl.ds(j * LANES, LANES)
                p = (plsc.bitcast(r0_v[b, sl], jnp.float32)
                     * plsc.bitcast(r1_v[b, sl], jnp.float32)
                     * plsc.bitcast(r2_v[b, sl], jnp.float32))
                acc = p if acc is None else acc + p
            vec = jnp.where(lane_iota == jnp.int32(l), jnp.sum(acc), vec)
        out_v[pl.ds(gbase, LANES)] = vec
        return carry

    lax.fori_loop(jnp.int32(0), jnp.int32(BPW // LANES), body, jnp.int32(0))

    pltpu.sync_copy(out_v, out_hbm.at[pl.ds(base, BPW)])


@jax.jit
def _run(f0, f1, f2, i0, i1, i2):
    mesh = plsc.VectorSubcoreMesh(core_axis_name="c", subcore_axis_name="s")
    kern = functools.partial(
        pl.kernel,
        out_type=jax.ShapeDtypeStruct((B,), jnp.float32),
        mesh=mesh,
        scratch_types=[
            pltpu.VMEM((BPW,), jnp.int32),
            pltpu.VMEM((BPW,), jnp.int32),
            pltpu.VMEM((BPW,), jnp.int32),
            pltpu.VMEM((BPW, K), jnp.int32),
            pltpu.VMEM((BPW, K), jnp.int32),
            pltpu.VMEM((BPW, K), jnp.int32),
            pltpu.VMEM((BPW,), jnp.float32),
            pltpu.SemaphoreType.DMA,
        ],
        compiler_params=pltpu.CompilerParams(
            needs_layout_passes=False, use_tc_tiling_on_sc=False),
    )(_sc_kernel_body)
    return kern(f0, f1, f2, i0, i1, i2)


def _hi_plane(f):
    from jax.experimental import layout as jex_layout
    hi = f.astype(jnp.float32)
    hi = jex_layout.with_layout_constraint(hi, jex_layout.Layout((1, 0)))
    return lax.bitcast_convert_type(hi, jnp.int32)


def kernel(indices, F0, F1, F2):
    idx = indices.astype(jnp.int32)
    out = _run(_hi_plane(F0), _hi_plane(F1), _hi_plane(F2),
               idx[:, 0], idx[:, 1], idx[:, 2])
    return out.astype(jnp.float64)
